# Initial kernel scaffold; baseline (speedup 1.0000x reference)
import jax
import jax.numpy as jnp
from jax.experimental import pallas as pl


def kernel(x, edge_index, W1, b1, gamma, beta, W2, b2):
    raise NotImplementedError("write your pallas kernel here")


# speedup vs baseline: 20.3413x; 20.3413x over previous
import jax
import jax.numpy as jnp
from jax import lax
from jax.experimental import pallas as pl
from jax.experimental.pallas import tpu as pltpu
from jax.experimental.pallas import tpu_sc as plsc

N = 10000
E = 320000
D = 128

NC = 2
NS = 16
NW = NC * NS
EPW = E // NW
NP = 10240
RPS = NP // NS
HP = 128

KE = 128
EPWP = NP
PADW = EPWP - EPW
CHW = EPWP // KE

_mesh = plsc.VectorSubcoreMesh(core_axis_name="c", subcore_axis_name="s")


def _hist_body(dst_hbm, zeros_hbm, ones_hbm, out_hbm,
               hist_sh, didx0, didx1, ones_v, sem0, sem1):
    cid = lax.axis_index("c")
    sid = lax.axis_index("s")
    wid = cid * NS + sid
    pltpu.sync_copy(ones_hbm, ones_v)
    pltpu.sync_copy(dst_hbm.at[wid, 0], didx0)
    pltpu.sync_copy(dst_hbm.at[wid, 1], didx1)
    pltpu.sync_copy(zeros_hbm.at[pl.ds(sid * RPS, RPS)],
                    hist_sh.at[pl.ds(sid * RPS, RPS)])
    plsc.subcore_barrier()
    pltpu.async_copy(ones_v, hist_sh.at[didx0], sem0, add=True)
    pltpu.async_copy(ones_v, hist_sh.at[didx1], sem1, add=True)

    def pair(g, carry):
        c0 = 2 * g
        pltpu.make_async_copy(ones_v, hist_sh.at[didx0], sem0).wait()
        pltpu.sync_copy(dst_hbm.at[wid, c0 + 2], didx0)
        pltpu.async_copy(ones_v, hist_sh.at[didx0], sem0, add=True)
        pltpu.make_async_copy(ones_v, hist_sh.at[didx1], sem1).wait()
        pltpu.sync_copy(dst_hbm.at[wid, c0 + 3], didx1)
        pltpu.async_copy(ones_v, hist_sh.at[didx1], sem1, add=True)
        return carry

    lax.fori_loop(0, CHW // 2 - 1, pair, 0)
    pltpu.make_async_copy(ones_v, hist_sh.at[didx0], sem0).wait()
    pltpu.make_async_copy(ones_v, hist_sh.at[didx1], sem1).wait()
    plsc.subcore_barrier()
    pltpu.sync_copy(hist_sh.at[pl.ds(sid * RPS, RPS)],
                    out_hbm.at[cid, pl.ds(sid * RPS, RPS)])


_hist = pl.kernel(
    _hist_body,
    out_type=jax.ShapeDtypeStruct((NC, NP, HP), jnp.float32),
    mesh=_mesh,
    scratch_types=[
        pltpu.VMEM_SHARED((NP, HP), jnp.float32),
        pltpu.VMEM((KE,), jnp.int32),
        pltpu.VMEM((KE,), jnp.int32),
        pltpu.VMEM((KE, HP), jnp.float32),
        pltpu.SemaphoreType.DMA,
        pltpu.SemaphoreType.DMA,
    ],
)


def _scatter_body(rows_hbm, src_hbm, dst_hbm, zeros_hbm, out_hbm,
                  acc_sh, sidx0, sidx1, didx0, didx1,
                  rows_v0, rows_v1, sem0, sem1):
    cid = lax.axis_index("c")
    sid = lax.axis_index("s")
    wid = cid * NS + sid
    pltpu.sync_copy(src_hbm.at[wid, 0], sidx0)
    pltpu.sync_copy(dst_hbm.at[wid, 0], didx0)
    pltpu.sync_copy(src_hbm.at[wid, 1], sidx1)
    pltpu.sync_copy(dst_hbm.at[wid, 1], didx1)
    pltpu.async_copy(rows_hbm.at[sidx0], rows_v0, sem0)
    pltpu.async_copy(rows_hbm.at[sidx1], rows_v1, sem1)
    pltpu.sync_copy(zeros_hbm.at[pl.ds(sid * RPS, RPS)],
                    acc_sh.at[pl.ds(sid * RPS, RPS)])
    plsc.subcore_barrier()

    def pair(g, carry):
        c0 = 2 * g
        pltpu.make_async_copy(rows_hbm.at[sidx0], rows_v0, sem0).wait()
        pltpu.sync_copy(rows_v0, acc_sh.at[didx0], add=True)
        pltpu.sync_copy(src_hbm.at[wid, c0 + 2], sidx0)
        pltpu.sync_copy(dst_hbm.at[wid, c0 + 2], didx0)
        pltpu.async_copy(rows_hbm.at[sidx0], rows_v0, sem0)

        pltpu.make_async_copy(rows_hbm.at[sidx1], rows_v1, sem1).wait()
        pltpu.sync_copy(rows_v1, acc_sh.at[didx1], add=True)
        pltpu.sync_copy(src_hbm.at[wid, c0 + 3], sidx1)
        pltpu.sync_copy(dst_hbm.at[wid, c0 + 3], didx1)
        pltpu.async_copy(rows_hbm.at[sidx1], rows_v1, sem1)
        return carry

    lax.fori_loop(0, CHW // 2 - 1, pair, 0)
    pltpu.make_async_copy(rows_hbm.at[sidx0], rows_v0, sem0).wait()
    pltpu.sync_copy(rows_v0, acc_sh.at[didx0], add=True)
    pltpu.make_async_copy(rows_hbm.at[sidx1], rows_v1, sem1).wait()
    pltpu.sync_copy(rows_v1, acc_sh.at[didx1], add=True)

    plsc.subcore_barrier()
    pltpu.sync_copy(acc_sh.at[pl.ds(sid * RPS, RPS)],
                    out_hbm.at[cid, pl.ds(sid * RPS, RPS)])


_scatter = pl.kernel(
    _scatter_body,
    out_type=jax.ShapeDtypeStruct((NC, NP, D), jnp.float32),
    mesh=_mesh,
    scratch_types=[
        pltpu.VMEM_SHARED((NP, D), jnp.float32),
        pltpu.VMEM((KE,), jnp.int32),
        pltpu.VMEM((KE,), jnp.int32),
        pltpu.VMEM((KE,), jnp.int32),
        pltpu.VMEM((KE,), jnp.int32),
        pltpu.VMEM((KE, D), jnp.float32),
        pltpu.VMEM((KE, D), jnp.float32),
        pltpu.SemaphoreType.DMA,
        pltpu.SemaphoreType.DMA,
    ],
)

R = 1000


def _dinv_rows(hist_ref):
    deg = hist_ref[0, :, 0] + hist_ref[1, :, 0] + 1.0
    return lax.rsqrt(deg)[:, None]


def _tc1_body(x_ref, w_ref, hist_ref, out_ref):
    h = jnp.dot(x_ref[...], w_ref[...], preferred_element_type=jnp.float32)
    out_ref[...] = h * _dinv_rows(hist_ref)


_tc1 = pl.pallas_call(
    _tc1_body,
    grid=(N // R,),
    in_specs=[
        pl.BlockSpec((R, D), lambda i: (i, 0)),
        pl.BlockSpec((D, D), lambda i: (0, 0)),
        pl.BlockSpec((NC, R, HP), lambda i: (0, i, 0)),
    ],
    out_specs=pl.BlockSpec((R, D), lambda i: (i, 0)),
    out_shape=jax.ShapeDtypeStruct((N, D), jnp.float32),
)


def _tc2_body(accp_ref, t1p_ref, hist_ref, b1_ref, gamma_ref, beta_ref,
              w2_ref, out_ref):
    dinv = _dinv_rows(hist_ref)
    acc = accp_ref[0] + accp_ref[1] + t1p_ref[...]
    conv = acc * dinv + b1_ref[...]
    s = 1.0 / jnp.sqrt(1.0 + 1e-5)
    h = jnp.maximum(conv * (gamma_ref[...] * s) + beta_ref[...], 0.0)
    out_ref[...] = jnp.dot(h, w2_ref[...],
                           preferred_element_type=jnp.float32) * dinv


_tc2 = pl.pallas_call(
    _tc2_body,
    grid=(N // R,),
    in_specs=[
        pl.BlockSpec((NC, R, D), lambda i: (0, i, 0)),
        pl.BlockSpec((R, D), lambda i: (i, 0)),
        pl.BlockSpec((NC, R, HP), lambda i: (0, i, 0)),
        pl.BlockSpec((1, D), lambda i: (0, 0)),
        pl.BlockSpec((1, D), lambda i: (0, 0)),
        pl.BlockSpec((1, D), lambda i: (0, 0)),
        pl.BlockSpec((D, D), lambda i: (0, 0)),
    ],
    out_specs=pl.BlockSpec((R, D), lambda i: (i, 0)),
    out_shape=jax.ShapeDtypeStruct((N, D), jnp.float32),
)


def _tc3_body(accp_ref, t2p_ref, hist_ref, b2_ref, x_ref, out_ref):
    dinv = _dinv_rows(hist_ref)
    acc = accp_ref[0] + accp_ref[1] + t2p_ref[...]
    out_ref[...] = acc * dinv + b2_ref[...] + x_ref[...]


_tc3 = pl.pallas_call(
    _tc3_body,
    grid=(N // R,),
    in_specs=[
        pl.BlockSpec((NC, R, D), lambda i: (0, i, 0)),
        pl.BlockSpec((R, D), lambda i: (i, 0)),
        pl.BlockSpec((NC, R, HP), lambda i: (0, i, 0)),
        pl.BlockSpec((1, D), lambda i: (0, 0)),
        pl.BlockSpec((R, D), lambda i: (i, 0)),
    ],
    out_specs=pl.BlockSpec((R, D), lambda i: (i, 0)),
    out_shape=jax.ShapeDtypeStruct((N, D), jnp.float32),
)


def kernel(x, edge_index, W1, b1, gamma, beta, W2, b2):
    src2 = edge_index[0].reshape(NW, EPW)
    dst2 = edge_index[1].reshape(NW, EPW)
    pad_s = jnp.broadcast_to(jnp.arange(PADW, dtype=jnp.int32)[None],
                             (NW, PADW))
    pad_d = jnp.broadcast_to((N + jnp.arange(PADW, dtype=jnp.int32))[None],
                             (NW, PADW))
    srcp = jnp.concatenate([src2, pad_s], axis=1).reshape(NW, CHW, KE)
    dstp = jnp.concatenate([dst2, pad_d], axis=1).reshape(NW, CHW, KE)

    zeros_nd = jnp.zeros((NP, D), jnp.float32)
    zeros_nh = jnp.zeros((NP, HP), jnp.float32)
    ones_kh = jnp.ones((KE, HP), jnp.float32)

    hist = _hist(dstp, zeros_nh, ones_kh)
    t1p = _tc1(x, W1, hist)
    acc1 = _scatter(t1p, srcp, dstp, zeros_nd)
    t2p = _tc2(acc1, t1p, hist, b1.reshape(1, D), gamma.reshape(1, D),
               beta.reshape(1, D), W2)
    acc2 = _scatter(t2p, srcp, dstp, zeros_nd)
    out = _tc3(acc2, t2p, hist, b2.reshape(1, D), x)
    return out


# speedup vs baseline: 22.4895x; 1.1056x over previous
import jax
import jax.numpy as jnp
from jax import lax
from jax.experimental import pallas as pl
from jax.experimental.pallas import tpu as pltpu
from jax.experimental.pallas import tpu_sc as plsc

N = 10000
E = 320000
D = 128

NC = 2
NS = 16
NW = NC * NS
EPW = E // NW
NP = 10240
RPS = NP // NS
HP = 128

KE = 128
EPWP = NP
PADW = EPWP - EPW
CHW = EPWP // KE

_mesh = plsc.VectorSubcoreMesh(core_axis_name="c", subcore_axis_name="s")


def _hist_body(dst_hbm, zeros_hbm, ones_hbm, out_hbm,
               hist_sh, didx_v, ones_v, sem0, sem1):
    cid = lax.axis_index("c")
    sid = lax.axis_index("s")
    wid = cid * NS + sid
    pltpu.sync_copy(ones_hbm, ones_v)
    pltpu.sync_copy(dst_hbm.at[wid], didx_v)
    pltpu.sync_copy(zeros_hbm.at[pl.ds(sid * RPS, RPS)],
                    hist_sh.at[pl.ds(sid * RPS, RPS)])
    plsc.subcore_barrier()
    pltpu.async_copy(ones_v, hist_sh.at[didx_v.at[0]], sem0, add=True)
    pltpu.async_copy(ones_v, hist_sh.at[didx_v.at[1]], sem1, add=True)

    def pair(g, carry):
        c0 = 2 * g
        pltpu.make_async_copy(ones_v, hist_sh.at[didx_v.at[c0]],
                              sem0).wait()
        pltpu.async_copy(ones_v, hist_sh.at[didx_v.at[c0 + 2]], sem0,
                         add=True)
        pltpu.make_async_copy(ones_v, hist_sh.at[didx_v.at[c0 + 1]],
                              sem1).wait()
        pltpu.async_copy(ones_v, hist_sh.at[didx_v.at[c0 + 3]], sem1,
                         add=True)
        return carry

    lax.fori_loop(0, CHW // 2 - 1, pair, 0)
    pltpu.make_async_copy(ones_v, hist_sh.at[didx_v.at[CHW - 2]],
                          sem0).wait()
    pltpu.make_async_copy(ones_v, hist_sh.at[didx_v.at[CHW - 1]],
                          sem1).wait()
    plsc.subcore_barrier()
    pltpu.sync_copy(hist_sh.at[pl.ds(sid * RPS, RPS)],
                    out_hbm.at[cid, pl.ds(sid * RPS, RPS)])


_hist = pl.kernel(
    _hist_body,
    out_type=jax.ShapeDtypeStruct((NC, NP, HP), jnp.float32),
    mesh=_mesh,
    scratch_types=[
        pltpu.VMEM_SHARED((NP, HP), jnp.float32),
        pltpu.VMEM((CHW, KE), jnp.int32),
        pltpu.VMEM((KE, HP), jnp.float32),
        pltpu.SemaphoreType.DMA,
        pltpu.SemaphoreType.DMA,
    ],
)


def _scatter_body(rows_hbm, src_hbm, dst_hbm, zeros_hbm, out_hbm,
                  acc_sh, sidx0, sidx1, didx0, didx1, rows_v0, rows_v1,
                  semr0, semr1, semi0, semi1):
    cid = lax.axis_index("c")
    sid = lax.axis_index("s")
    wid = cid * NS + sid
    pltpu.sync_copy(src_hbm.at[wid, 0], sidx0)
    pltpu.sync_copy(dst_hbm.at[wid, 0], didx0)
    pltpu.async_copy(rows_hbm.at[sidx0], rows_v0, semr0)
    pltpu.async_copy(src_hbm.at[wid, 1], sidx1, semi1)
    pltpu.async_copy(dst_hbm.at[wid, 1], didx1, semi1)
    pltpu.sync_copy(zeros_hbm.at[pl.ds(sid * RPS, RPS)],
                    acc_sh.at[pl.ds(sid * RPS, RPS)])
    plsc.subcore_barrier()

    def pair(g, carry):
        c0 = 2 * g
        pltpu.make_async_copy(rows_hbm.at[sidx0], rows_v0, semr0).wait()
        pltpu.make_async_copy(src_hbm.at[wid, 0], sidx1, semi1).wait()
        pltpu.make_async_copy(dst_hbm.at[wid, 0], didx1, semi1).wait()
        pltpu.async_copy(rows_hbm.at[sidx1], rows_v1, semr1)
        pltpu.sync_copy(rows_v0, acc_sh.at[didx0], add=True)
        pltpu.async_copy(src_hbm.at[wid, c0 + 2], sidx0, semi0)
        pltpu.async_copy(dst_hbm.at[wid, c0 + 2], didx0, semi0)

        pltpu.make_async_copy(rows_hbm.at[sidx1], rows_v1, semr1).wait()
        pltpu.make_async_copy(src_hbm.at[wid, 0], sidx0, semi0).wait()
        pltpu.make_async_copy(dst_hbm.at[wid, 0], didx0, semi0).wait()
        pltpu.async_copy(rows_hbm.at[sidx0], rows_v0, semr0)
        pltpu.sync_copy(rows_v1, acc_sh.at[didx1], add=True)
        pltpu.async_copy(src_hbm.at[wid, c0 + 3], sidx1, semi1)
        pltpu.async_copy(dst_hbm.at[wid, c0 + 3], didx1, semi1)
        return carry

    lax.fori_loop(0, CHW // 2 - 1, pair, 0)
    pltpu.make_async_copy(rows_hbm.at[sidx0], rows_v0, semr0).wait()
    pltpu.make_async_copy(src_hbm.at[wid, 0], sidx1, semi1).wait()
    pltpu.make_async_copy(dst_hbm.at[wid, 0], didx1, semi1).wait()
    pltpu.async_copy(rows_hbm.at[sidx1], rows_v1, semr1)
    pltpu.sync_copy(rows_v0, acc_sh.at[didx0], add=True)
    pltpu.make_async_copy(rows_hbm.at[sidx1], rows_v1, semr1).wait()
    pltpu.sync_copy(rows_v1, acc_sh.at[didx1], add=True)

    plsc.subcore_barrier()
    pltpu.sync_copy(acc_sh.at[pl.ds(sid * RPS, RPS)],
                    out_hbm.at[cid, pl.ds(sid * RPS, RPS)])


_scatter = pl.kernel(
    _scatter_body,
    out_type=jax.ShapeDtypeStruct((NC, NP, D), jnp.float32),
    mesh=_mesh,
    scratch_types=[
        pltpu.VMEM_SHARED((NP, D), jnp.float32),
        pltpu.VMEM((KE,), jnp.int32),
        pltpu.VMEM((KE,), jnp.int32),
        pltpu.VMEM((KE,), jnp.int32),
        pltpu.VMEM((KE,), jnp.int32),
        pltpu.VMEM((KE, D), jnp.float32),
        pltpu.VMEM((KE, D), jnp.float32),
        pltpu.SemaphoreType.DMA,
        pltpu.SemaphoreType.DMA,
        pltpu.SemaphoreType.DMA,
        pltpu.SemaphoreType.DMA,
    ],
)

R = 1000


def _dinv_rows(hist_ref):
    deg = hist_ref[0, :, 0] + hist_ref[1, :, 0] + 1.0
    return lax.rsqrt(deg)[:, None]


def _tc1_body(x_ref, w_ref, hist_ref, out_ref):
    h = jnp.dot(x_ref[...], w_ref[...], preferred_element_type=jnp.float32)
    out_ref[...] = h * _dinv_rows(hist_ref)


_tc1 = pl.pallas_call(
    _tc1_body,
    grid=(N // R,),
    in_specs=[
        pl.BlockSpec((R, D), lambda i: (i, 0)),
        pl.BlockSpec((D, D), lambda i: (0, 0)),
        pl.BlockSpec((NC, R, HP), lambda i: (0, i, 0)),
    ],
    out_specs=pl.BlockSpec((R, D), lambda i: (i, 0)),
    out_shape=jax.ShapeDtypeStruct((N, D), jnp.float32),
)


def _tc2_body(accp_ref, t1p_ref, hist_ref, b1_ref, gamma_ref, beta_ref,
              w2_ref, out_ref):
    dinv = _dinv_rows(hist_ref)
    acc = accp_ref[0] + accp_ref[1] + t1p_ref[...]
    conv = acc * dinv + b1_ref[...]
    s = 1.0 / jnp.sqrt(1.0 + 1e-5)
    h = jnp.maximum(conv * (gamma_ref[...] * s) + beta_ref[...], 0.0)
    out_ref[...] = jnp.dot(h, w2_ref[...],
                           preferred_element_type=jnp.float32) * dinv


_tc2 = pl.pallas_call(
    _tc2_body,
    grid=(N // R,),
    in_specs=[
        pl.BlockSpec((NC, R, D), lambda i: (0, i, 0)),
        pl.BlockSpec((R, D), lambda i: (i, 0)),
        pl.BlockSpec((NC, R, HP), lambda i: (0, i, 0)),
        pl.BlockSpec((1, D), lambda i: (0, 0)),
        pl.BlockSpec((1, D), lambda i: (0, 0)),
        pl.BlockSpec((1, D), lambda i: (0, 0)),
        pl.BlockSpec((D, D), lambda i: (0, 0)),
    ],
    out_specs=pl.BlockSpec((R, D), lambda i: (i, 0)),
    out_shape=jax.ShapeDtypeStruct((N, D), jnp.float32),
)


def _tc3_body(accp_ref, t2p_ref, hist_ref, b2_ref, x_ref, out_ref):
    dinv = _dinv_rows(hist_ref)
    acc = accp_ref[0] + accp_ref[1] + t2p_ref[...]
    out_ref[...] = acc * dinv + b2_ref[...] + x_ref[...]


_tc3 = pl.pallas_call(
    _tc3_body,
    grid=(N // R,),
    in_specs=[
        pl.BlockSpec((NC, R, D), lambda i: (0, i, 0)),
        pl.BlockSpec((R, D), lambda i: (i, 0)),
        pl.BlockSpec((NC, R, HP), lambda i: (0, i, 0)),
        pl.BlockSpec((1, D), lambda i: (0, 0)),
        pl.BlockSpec((R, D), lambda i: (i, 0)),
    ],
    out_specs=pl.BlockSpec((R, D), lambda i: (i, 0)),
    out_shape=jax.ShapeDtypeStruct((N, D), jnp.float32),
)


def kernel(x, edge_index, W1, b1, gamma, beta, W2, b2):
    src2 = edge_index[0].reshape(NW, EPW)
    dst2 = edge_index[1].reshape(NW, EPW)
    pad_s = jnp.broadcast_to(jnp.arange(PADW, dtype=jnp.int32)[None],
                             (NW, PADW))
    pad_d = jnp.broadcast_to((N + jnp.arange(PADW, dtype=jnp.int32))[None],
                             (NW, PADW))
    srcp = jnp.concatenate([src2, pad_s], axis=1).reshape(NW, CHW, KE)
    dstp = jnp.concatenate([dst2, pad_d], axis=1).reshape(NW, CHW, KE)

    zeros_nd = jnp.zeros((NP, D), jnp.float32)
    zeros_nh = jnp.zeros((NP, HP), jnp.float32)
    ones_kh = jnp.ones((KE, HP), jnp.float32)

    hist = _hist(dstp, zeros_nh, ones_kh)
    t1p = _tc1(x, W1, hist)
    acc1 = _scatter(t1p, srcp, dstp, zeros_nd)
    t2p = _tc2(acc1, t1p, hist, b1.reshape(1, D), gamma.reshape(1, D),
               beta.reshape(1, D), W2)
    acc2 = _scatter(t2p, srcp, dstp, zeros_nd)
    out = _tc3(acc2, t2p, hist, b2.reshape(1, D), x)
    return out


# speedup vs baseline: 22.5889x; 1.0044x over previous
import jax
import jax.numpy as jnp
from jax import lax
from jax.experimental import pallas as pl
from jax.experimental.pallas import tpu as pltpu
from jax.experimental.pallas import tpu_sc as plsc

N = 10000
E = 320000
D = 128

NC = 2
NS = 16
NW = NC * NS
EPW = E // NW
NP = 10240
RPS = NP // NS
HP = 128

KE = 128
EPWP = NP
PADW = EPWP - EPW
CHW = EPWP // KE

_mesh = plsc.VectorSubcoreMesh(core_axis_name="c", subcore_axis_name="s")


def _hist_body(dst_hbm, zeros_hbm, ones_hbm, out_hbm,
               hist_sh, didx_v, ones_v, sem0, sem1):
    cid = lax.axis_index("c")
    sid = lax.axis_index("s")
    wid = cid * NS + sid
    pltpu.sync_copy(ones_hbm, ones_v)
    pltpu.sync_copy(dst_hbm.at[wid], didx_v)
    pltpu.sync_copy(zeros_hbm.at[pl.ds(sid * RPS, RPS)],
                    hist_sh.at[pl.ds(sid * RPS, RPS)])
    plsc.subcore_barrier()
    pltpu.async_copy(ones_v, hist_sh.at[didx_v.at[0]], sem0, add=True)
    pltpu.async_copy(ones_v, hist_sh.at[didx_v.at[1]], sem1, add=True)

    def pair(g, carry):
        c0 = 2 * g
        pltpu.make_async_copy(ones_v, hist_sh.at[didx_v.at[c0]],
                              sem0).wait()
        pltpu.async_copy(ones_v, hist_sh.at[didx_v.at[c0 + 2]], sem0,
                         add=True)
        pltpu.make_async_copy(ones_v, hist_sh.at[didx_v.at[c0 + 1]],
                              sem1).wait()
        pltpu.async_copy(ones_v, hist_sh.at[didx_v.at[c0 + 3]], sem1,
                         add=True)
        return carry

    lax.fori_loop(0, CHW // 2 - 1, pair, 0)
    pltpu.make_async_copy(ones_v, hist_sh.at[didx_v.at[CHW - 2]],
                          sem0).wait()
    pltpu.make_async_copy(ones_v, hist_sh.at[didx_v.at[CHW - 1]],
                          sem1).wait()
    plsc.subcore_barrier()
    pltpu.sync_copy(hist_sh.at[pl.ds(sid * RPS, RPS)],
                    out_hbm.at[cid, pl.ds(sid * RPS, RPS)])


_hist = pl.kernel(
    _hist_body,
    out_type=jax.ShapeDtypeStruct((NC, NP, HP), jnp.float32),
    mesh=_mesh,
    scratch_types=[
        pltpu.VMEM_SHARED((NP, HP), jnp.float32),
        pltpu.VMEM((CHW, KE), jnp.int32),
        pltpu.VMEM((KE, HP), jnp.float32),
        pltpu.SemaphoreType.DMA,
        pltpu.SemaphoreType.DMA,
    ],
)


def _scatter_body(rows_hbm, sd_hbm, zeros_hbm, out_hbm,
                  acc_sh, idx0, idx1, idx2, idx3, rows_v0, rows_v1,
                  semr0, semr1, semw0, semw1,
                  semi0, semi1, semi2, semi3):
    cid = lax.axis_index("c")
    sid = lax.axis_index("s")
    wid = cid * NS + sid
    idx4 = (idx0, idx1, idx2, idx3)
    rows = (rows_v0, rows_v1)
    semr = (semr0, semr1)
    semw = (semw0, semw1)
    semi = (semi0, semi1, semi2, semi3)

    def slot(c, f, p, do_w4, do_pref, do_gather):
        q, f1, f2 = 1 - p, (f + 1) % 4, (f + 2) % 4
        pltpu.make_async_copy(rows_hbm.at[idx4[f].at[0]], rows[p],
                              semr[p]).wait()
        pltpu.async_copy(rows[p], acc_sh.at[idx4[f].at[1]], semw[p],
                         add=True)
        if do_gather:
            pltpu.make_async_copy(sd_hbm.at[wid, 0], idx4[f1],
                                  semi[f1]).wait()
            if do_w4:
                pltpu.make_async_copy(rows[q], acc_sh.at[idx4[f1].at[1]],
                                      semw[q]).wait()
            pltpu.async_copy(rows_hbm.at[idx4[f1].at[0]], rows[q], semr[q])
        if do_pref:
            pltpu.async_copy(sd_hbm.at[wid, c + 2], idx4[f2], semi[f2])

    pltpu.sync_copy(sd_hbm.at[wid, 0], idx0)
    pltpu.async_copy(rows_hbm.at[idx0.at[0]], rows_v0, semr0)
    pltpu.async_copy(sd_hbm.at[wid, 1], idx1, semi1)
    pltpu.sync_copy(zeros_hbm.at[pl.ds(sid * RPS, RPS)],
                    acc_sh.at[pl.ds(sid * RPS, RPS)])
    plsc.subcore_barrier()

    slot(0, 0, 0, False, True, True)
    slot(1, 1, 1, True, True, True)
    slot(2, 2, 0, True, True, True)
    slot(3, 3, 1, True, True, True)

    def quad(g, carry):
        c0 = 4 * g
        slot(c0 + 0, 0, 0, True, True, True)
        slot(c0 + 1, 1, 1, True, True, True)
        slot(c0 + 2, 2, 0, True, True, True)
        slot(c0 + 3, 3, 1, True, True, True)
        return carry

    lax.fori_loop(1, CHW // 4 - 1, quad, 0)
    slot(CHW - 4, 0, 0, True, True, True)
    slot(CHW - 3, 1, 1, True, True, True)
    slot(CHW - 2, 2, 0, True, False, True)
    slot(CHW - 1, 3, 1, True, False, False)
    pltpu.make_async_copy(rows_v0, acc_sh.at[idx2.at[1]], semw0).wait()
    pltpu.make_async_copy(rows_v1, acc_sh.at[idx3.at[1]], semw1).wait()

    plsc.subcore_barrier()
    pltpu.sync_copy(acc_sh.at[pl.ds(sid * RPS, RPS)],
                    out_hbm.at[cid, pl.ds(sid * RPS, RPS)])


_scatter = pl.kernel(
    _scatter_body,
    out_type=jax.ShapeDtypeStruct((NC, NP, D), jnp.float32),
    mesh=_mesh,
    scratch_types=[
        pltpu.VMEM_SHARED((NP, D), jnp.float32),
        pltpu.VMEM((2, KE), jnp.int32),
        pltpu.VMEM((2, KE), jnp.int32),
        pltpu.VMEM((2, KE), jnp.int32),
        pltpu.VMEM((2, KE), jnp.int32),
        pltpu.VMEM((KE, D), jnp.float32),
        pltpu.VMEM((KE, D), jnp.float32),
        pltpu.SemaphoreType.DMA,
        pltpu.SemaphoreType.DMA,
        pltpu.SemaphoreType.DMA,
        pltpu.SemaphoreType.DMA,
        pltpu.SemaphoreType.DMA,
        pltpu.SemaphoreType.DMA,
        pltpu.SemaphoreType.DMA,
        pltpu.SemaphoreType.DMA,
    ],
)

R = 1000


def _dinv_rows(hist_ref):
    deg = hist_ref[0, :, 0] + hist_ref[1, :, 0] + 1.0
    return lax.rsqrt(deg)[:, None]


def _tc1_body(x_ref, w_ref, hist_ref, out_ref):
    h = jnp.dot(x_ref[...], w_ref[...], preferred_element_type=jnp.float32)
    out_ref[...] = h * _dinv_rows(hist_ref)


_tc1 = pl.pallas_call(
    _tc1_body,
    grid=(N // R,),
    in_specs=[
        pl.BlockSpec((R, D), lambda i: (i, 0)),
        pl.BlockSpec((D, D), lambda i: (0, 0)),
        pl.BlockSpec((NC, R, HP), lambda i: (0, i, 0)),
    ],
    out_specs=pl.BlockSpec((R, D), lambda i: (i, 0)),
    out_shape=jax.ShapeDtypeStruct((N, D), jnp.float32),
)


def _tc2_body(accp_ref, t1p_ref, hist_ref, b1_ref, gamma_ref, beta_ref,
              w2_ref, out_ref):
    dinv = _dinv_rows(hist_ref)
    acc = accp_ref[0] + accp_ref[1] + t1p_ref[...]
    conv = acc * dinv + b1_ref[...]
    s = 1.0 / jnp.sqrt(1.0 + 1e-5)
    h = jnp.maximum(conv * (gamma_ref[...] * s) + beta_ref[...], 0.0)
    out_ref[...] = jnp.dot(h, w2_ref[...],
                           preferred_element_type=jnp.float32) * dinv


_tc2 = pl.pallas_call(
    _tc2_body,
    grid=(N // R,),
    in_specs=[
        pl.BlockSpec((NC, R, D), lambda i: (0, i, 0)),
        pl.BlockSpec((R, D), lambda i: (i, 0)),
        pl.BlockSpec((NC, R, HP), lambda i: (0, i, 0)),
        pl.BlockSpec((1, D), lambda i: (0, 0)),
        pl.BlockSpec((1, D), lambda i: (0, 0)),
        pl.BlockSpec((1, D), lambda i: (0, 0)),
        pl.BlockSpec((D, D), lambda i: (0, 0)),
    ],
    out_specs=pl.BlockSpec((R, D), lambda i: (i, 0)),
    out_shape=jax.ShapeDtypeStruct((N, D), jnp.float32),
)


def _tc3_body(accp_ref, t2p_ref, hist_ref, b2_ref, x_ref, out_ref):
    dinv = _dinv_rows(hist_ref)
    acc = accp_ref[0] + accp_ref[1] + t2p_ref[...]
    out_ref[...] = acc * dinv + b2_ref[...] + x_ref[...]


_tc3 = pl.pallas_call(
    _tc3_body,
    grid=(N // R,),
    in_specs=[
        pl.BlockSpec((NC, R, D), lambda i: (0, i, 0)),
        pl.BlockSpec((R, D), lambda i: (i, 0)),
        pl.BlockSpec((NC, R, HP), lambda i: (0, i, 0)),
        pl.BlockSpec((1, D), lambda i: (0, 0)),
        pl.BlockSpec((R, D), lambda i: (i, 0)),
    ],
    out_specs=pl.BlockSpec((R, D), lambda i: (i, 0)),
    out_shape=jax.ShapeDtypeStruct((N, D), jnp.float32),
)


def kernel(x, edge_index, W1, b1, gamma, beta, W2, b2):
    src2 = edge_index[0].reshape(NW, EPW)
    dst2 = edge_index[1].reshape(NW, EPW)
    pad_s = jnp.broadcast_to(jnp.arange(PADW, dtype=jnp.int32)[None],
                             (NW, PADW))
    pad_d = jnp.broadcast_to((N + jnp.arange(PADW, dtype=jnp.int32))[None],
                             (NW, PADW))
    srcp = jnp.concatenate([src2, pad_s], axis=1).reshape(NW, CHW, KE)
    dstp = jnp.concatenate([dst2, pad_d], axis=1).reshape(NW, CHW, KE)
    sd = jnp.stack([srcp, dstp], axis=2)

    zeros_nd = jnp.zeros((NP, D), jnp.float32)
    zeros_nh = jnp.zeros((NP, HP), jnp.float32)
    ones_kh = jnp.ones((KE, HP), jnp.float32)

    hist = _hist(dstp, zeros_nh, ones_kh)
    t1p = _tc1(x, W1, hist)
    acc1 = _scatter(t1p, sd, zeros_nd)
    t2p = _tc2(acc1, t1p, hist, b1.reshape(1, D), gamma.reshape(1, D),
               beta.reshape(1, D), W2)
    acc2 = _scatter(t2p, sd, zeros_nd)
    out = _tc3(acc2, t2p, hist, b2.reshape(1, D), x)
    return out


# speedup vs baseline: 25.8597x; 1.1448x over previous
import jax
import jax.numpy as jnp
from jax import lax
from jax.experimental import pallas as pl
from jax.experimental.pallas import tpu as pltpu
from jax.experimental.pallas import tpu_sc as plsc

N = 10000
E = 320000
D = 128

NC = 2
NS = 16
NW = NC * NS
EPW = E // NW
NP = 10240
RPS = NP // NS
HP = 16

KE = 128
EPWP = NP
PADW = EPWP - EPW
CHW = EPWP // KE

_mesh = plsc.VectorSubcoreMesh(core_axis_name="c", subcore_axis_name="s")


def _hist_body(dst_hbm, zeros_hbm, ones_hbm, out_hbm,
               hist_sh, didx_v, ones_v, sem0, sem1):
    cid = lax.axis_index("c")
    sid = lax.axis_index("s")
    wid = cid * NS + sid
    pltpu.sync_copy(ones_hbm, ones_v)
    pltpu.sync_copy(dst_hbm.at[wid], didx_v)
    pltpu.sync_copy(zeros_hbm.at[pl.ds(sid * RPS, RPS)],
                    hist_sh.at[pl.ds(sid * RPS, RPS)])
    plsc.subcore_barrier()
    pltpu.async_copy(ones_v, hist_sh.at[didx_v.at[0]], sem0, add=True)
    pltpu.async_copy(ones_v, hist_sh.at[didx_v.at[1]], sem1, add=True)

    def pair(g, carry):
        c0 = 2 * g
        pltpu.make_async_copy(ones_v, hist_sh.at[didx_v.at[c0]],
                              sem0).wait()
        pltpu.async_copy(ones_v, hist_sh.at[didx_v.at[c0 + 2]], sem0,
                         add=True)
        pltpu.make_async_copy(ones_v, hist_sh.at[didx_v.at[c0 + 1]],
                              sem1).wait()
        pltpu.async_copy(ones_v, hist_sh.at[didx_v.at[c0 + 3]], sem1,
                         add=True)
        return carry

    lax.fori_loop(0, CHW // 2 - 1, pair, 0)
    pltpu.make_async_copy(ones_v, hist_sh.at[didx_v.at[CHW - 2]],
                          sem0).wait()
    pltpu.make_async_copy(ones_v, hist_sh.at[didx_v.at[CHW - 1]],
                          sem1).wait()
    plsc.subcore_barrier()
    pltpu.sync_copy(hist_sh.at[pl.ds(sid * RPS, RPS)],
                    out_hbm.at[cid, pl.ds(sid * RPS, RPS)])


_hist = pl.kernel(
    _hist_body,
    out_type=jax.ShapeDtypeStruct((NC, NP, HP), jnp.float32),
    mesh=_mesh,
    compiler_params=pltpu.CompilerParams(use_tc_tiling_on_sc=False),
    scratch_types=[
        pltpu.VMEM_SHARED((NP, HP), jnp.float32),
        pltpu.VMEM((CHW, KE), jnp.int32),
        pltpu.VMEM((KE, HP), jnp.float32),
        pltpu.SemaphoreType.DMA,
        pltpu.SemaphoreType.DMA,
    ],
)


def _scatter_body(rows_hbm, sd_hbm, zeros_hbm, out_hbm,
                  acc_sh, idx0, idx1, idx2, idx3, rows_v0, rows_v1,
                  semr0, semr1, semw0, semw1,
                  semi0, semi1, semi2, semi3):
    cid = lax.axis_index("c")
    sid = lax.axis_index("s")
    wid = cid * NS + sid
    idx4 = (idx0, idx1, idx2, idx3)
    rows = (rows_v0, rows_v1)
    semr = (semr0, semr1)
    semw = (semw0, semw1)
    semi = (semi0, semi1, semi2, semi3)

    def slot(c, f, p, do_w4, do_pref, do_gather):
        q, f1, f2 = 1 - p, (f + 1) % 4, (f + 2) % 4
        pltpu.make_async_copy(rows_hbm.at[idx4[f].at[0]], rows[p],
                              semr[p]).wait()
        pltpu.async_copy(rows[p], acc_sh.at[idx4[f].at[1]], semw[p],
                         add=True)
        if do_gather:
            pltpu.make_async_copy(sd_hbm.at[wid, 0], idx4[f1],
                                  semi[f1]).wait()
            if do_w4:
                pltpu.make_async_copy(rows[q], acc_sh.at[idx4[f1].at[1]],
                                      semw[q]).wait()
            pltpu.async_copy(rows_hbm.at[idx4[f1].at[0]], rows[q], semr[q])
        if do_pref:
            pltpu.async_copy(sd_hbm.at[wid, c + 2], idx4[f2], semi[f2])

    pltpu.sync_copy(sd_hbm.at[wid, 0], idx0)
    pltpu.async_copy(rows_hbm.at[idx0.at[0]], rows_v0, semr0)
    pltpu.async_copy(sd_hbm.at[wid, 1], idx1, semi1)
    pltpu.sync_copy(zeros_hbm.at[pl.ds(sid * RPS, RPS)],
                    acc_sh.at[pl.ds(sid * RPS, RPS)])
    plsc.subcore_barrier()

    slot(0, 0, 0, False, True, True)
    slot(1, 1, 1, True, True, True)
    slot(2, 2, 0, True, True, True)
    slot(3, 3, 1, True, True, True)

    def quad(g, carry):
        c0 = 4 * g
        slot(c0 + 0, 0, 0, True, True, True)
        slot(c0 + 1, 1, 1, True, True, True)
        slot(c0 + 2, 2, 0, True, True, True)
        slot(c0 + 3, 3, 1, True, True, True)
        return carry

    lax.fori_loop(1, CHW // 4 - 1, quad, 0)
    slot(CHW - 4, 0, 0, True, True, True)
    slot(CHW - 3, 1, 1, True, True, True)
    slot(CHW - 2, 2, 0, True, False, True)
    slot(CHW - 1, 3, 1, True, False, False)
    pltpu.make_async_copy(rows_v0, acc_sh.at[idx2.at[1]], semw0).wait()
    pltpu.make_async_copy(rows_v1, acc_sh.at[idx3.at[1]], semw1).wait()

    plsc.subcore_barrier()
    pltpu.sync_copy(acc_sh.at[pl.ds(sid * RPS, RPS)],
                    out_hbm.at[cid, pl.ds(sid * RPS, RPS)])


_scatter = pl.kernel(
    _scatter_body,
    out_type=jax.ShapeDtypeStruct((NC, NP, D), jnp.float32),
    mesh=_mesh,
    scratch_types=[
        pltpu.VMEM_SHARED((NP, D), jnp.float32),
        pltpu.VMEM((2, KE), jnp.int32),
        pltpu.VMEM((2, KE), jnp.int32),
        pltpu.VMEM((2, KE), jnp.int32),
        pltpu.VMEM((2, KE), jnp.int32),
        pltpu.VMEM((KE, D), jnp.float32),
        pltpu.VMEM((KE, D), jnp.float32),
        pltpu.SemaphoreType.DMA,
        pltpu.SemaphoreType.DMA,
        pltpu.SemaphoreType.DMA,
        pltpu.SemaphoreType.DMA,
        pltpu.SemaphoreType.DMA,
        pltpu.SemaphoreType.DMA,
        pltpu.SemaphoreType.DMA,
        pltpu.SemaphoreType.DMA,
    ],
)

R = 1000


def _dinv_rows(hist_ref):
    deg = hist_ref[0, :, 0] + hist_ref[1, :, 0] + 1.0
    return lax.rsqrt(deg)[:, None]


def _tc1_body(x_ref, w_ref, hist_ref, out_ref):
    h = jnp.dot(x_ref[...], w_ref[...], preferred_element_type=jnp.float32)
    out_ref[...] = h * _dinv_rows(hist_ref)


_tc1 = pl.pallas_call(
    _tc1_body,
    grid=(N // R,),
    in_specs=[
        pl.BlockSpec((R, D), lambda i: (i, 0)),
        pl.BlockSpec((D, D), lambda i: (0, 0)),
        pl.BlockSpec((NC, R, HP), lambda i: (0, i, 0)),
    ],
    out_specs=pl.BlockSpec((R, D), lambda i: (i, 0)),
    out_shape=jax.ShapeDtypeStruct((N, D), jnp.float32),
)


def _tc2_body(accp_ref, t1p_ref, hist_ref, b1_ref, gamma_ref, beta_ref,
              w2_ref, out_ref):
    dinv = _dinv_rows(hist_ref)
    acc = accp_ref[0] + accp_ref[1] + t1p_ref[...]
    conv = acc * dinv + b1_ref[...]
    s = 1.0 / jnp.sqrt(1.0 + 1e-5)
    h = jnp.maximum(conv * (gamma_ref[...] * s) + beta_ref[...], 0.0)
    out_ref[...] = jnp.dot(h, w2_ref[...],
                           preferred_element_type=jnp.float32) * dinv


_tc2 = pl.pallas_call(
    _tc2_body,
    grid=(N // R,),
    in_specs=[
        pl.BlockSpec((NC, R, D), lambda i: (0, i, 0)),
        pl.BlockSpec((R, D), lambda i: (i, 0)),
        pl.BlockSpec((NC, R, HP), lambda i: (0, i, 0)),
        pl.BlockSpec((1, D), lambda i: (0, 0)),
        pl.BlockSpec((1, D), lambda i: (0, 0)),
        pl.BlockSpec((1, D), lambda i: (0, 0)),
        pl.BlockSpec((D, D), lambda i: (0, 0)),
    ],
    out_specs=pl.BlockSpec((R, D), lambda i: (i, 0)),
    out_shape=jax.ShapeDtypeStruct((N, D), jnp.float32),
)


def _tc3_body(accp_ref, t2p_ref, hist_ref, b2_ref, x_ref, out_ref):
    dinv = _dinv_rows(hist_ref)
    acc = accp_ref[0] + accp_ref[1] + t2p_ref[...]
    out_ref[...] = acc * dinv + b2_ref[...] + x_ref[...]


_tc3 = pl.pallas_call(
    _tc3_body,
    grid=(N // R,),
    in_specs=[
        pl.BlockSpec((NC, R, D), lambda i: (0, i, 0)),
        pl.BlockSpec((R, D), lambda i: (i, 0)),
        pl.BlockSpec((NC, R, HP), lambda i: (0, i, 0)),
        pl.BlockSpec((1, D), lambda i: (0, 0)),
        pl.BlockSpec((R, D), lambda i: (i, 0)),
    ],
    out_specs=pl.BlockSpec((R, D), lambda i: (i, 0)),
    out_shape=jax.ShapeDtypeStruct((N, D), jnp.float32),
)


def kernel(x, edge_index, W1, b1, gamma, beta, W2, b2):
    src2 = edge_index[0].reshape(NW, EPW)
    dst2 = edge_index[1].reshape(NW, EPW)
    pad_s = jnp.broadcast_to(jnp.arange(PADW, dtype=jnp.int32)[None],
                             (NW, PADW))
    pad_d = jnp.broadcast_to((N + jnp.arange(PADW, dtype=jnp.int32))[None],
                             (NW, PADW))
    srcp = jnp.concatenate([src2, pad_s], axis=1).reshape(NW, CHW, KE)
    dstp = jnp.concatenate([dst2, pad_d], axis=1).reshape(NW, CHW, KE)
    sd = jnp.stack([srcp, dstp], axis=2)

    zeros_nd = jnp.zeros((NP, D), jnp.float32)
    zeros_nh = jnp.zeros((NP, HP), jnp.float32)
    ones_kh = jnp.ones((KE, HP), jnp.float32)

    hist = _hist(dstp, zeros_nh, ones_kh)
    t1p = _tc1(x, W1, hist)
    acc1 = _scatter(t1p, sd, zeros_nd)
    t2p = _tc2(acc1, t1p, hist, b1.reshape(1, D), gamma.reshape(1, D),
               beta.reshape(1, D), W2)
    acc2 = _scatter(t2p, sd, zeros_nd)
    out = _tc3(acc2, t2p, hist, b2.reshape(1, D), x)
    return out


# speedup vs baseline: 26.2160x; 1.0138x over previous
import jax
import jax.numpy as jnp
from jax import lax
from jax.experimental import pallas as pl
from jax.experimental.pallas import tpu as pltpu
from jax.experimental.pallas import tpu_sc as plsc

N = 10000
E = 320000
D = 128

NC = 2
NS = 16
NW = NC * NS
EPW = E // NW
NP = 10240
RPS = NP // NS
HP = 16

KE = 128
EPWP = NP
PADW = EPWP - EPW
CHW = EPWP // KE

_mesh = plsc.VectorSubcoreMesh(core_axis_name="c", subcore_axis_name="s")


def _hist_body(dst_hbm, zeros_hbm, ones_hbm, out_hbm,
               hist_sh, didx_v, ones_v, sem0, sem1):
    cid = lax.axis_index("c")
    sid = lax.axis_index("s")
    wid = cid * NS + sid
    pltpu.sync_copy(ones_hbm, ones_v)
    pltpu.sync_copy(dst_hbm.at[wid], didx_v)
    pltpu.sync_copy(zeros_hbm.at[pl.ds(sid * RPS, RPS)],
                    hist_sh.at[pl.ds(sid * RPS, RPS)])
    plsc.subcore_barrier()
    pltpu.async_copy(ones_v, hist_sh.at[didx_v.at[0]], sem0, add=True)
    pltpu.async_copy(ones_v, hist_sh.at[didx_v.at[1]], sem1, add=True)

    def pair(g, carry):
        c0 = 2 * g
        pltpu.make_async_copy(ones_v, hist_sh.at[didx_v.at[c0]],
                              sem0).wait()
        pltpu.async_copy(ones_v, hist_sh.at[didx_v.at[c0 + 2]], sem0,
                         add=True)
        pltpu.make_async_copy(ones_v, hist_sh.at[didx_v.at[c0 + 1]],
                              sem1).wait()
        pltpu.async_copy(ones_v, hist_sh.at[didx_v.at[c0 + 3]], sem1,
                         add=True)
        return carry

    lax.fori_loop(0, CHW // 2 - 1, pair, 0)
    pltpu.make_async_copy(ones_v, hist_sh.at[didx_v.at[CHW - 2]],
                          sem0).wait()
    pltpu.make_async_copy(ones_v, hist_sh.at[didx_v.at[CHW - 1]],
                          sem1).wait()
    plsc.subcore_barrier()
    pltpu.sync_copy(hist_sh.at[pl.ds(sid * RPS, RPS)],
                    out_hbm.at[cid, pl.ds(sid * RPS, RPS)])


_hist = pl.kernel(
    _hist_body,
    out_type=jax.ShapeDtypeStruct((NC, NP, HP), jnp.float32),
    mesh=_mesh,
    compiler_params=pltpu.CompilerParams(use_tc_tiling_on_sc=False),
    scratch_types=[
        pltpu.VMEM_SHARED((NP, HP), jnp.float32),
        pltpu.VMEM((CHW, KE), jnp.int32),
        pltpu.VMEM((KE, HP), jnp.float32),
        pltpu.SemaphoreType.DMA,
        pltpu.SemaphoreType.DMA,
    ],
)


def _scatter_body(rows_hbm, sd_hbm, zeros_hbm, out_hbm,
                  acc_sh, idx0, idx1, idx2, idx3, rows_v0, rows_v1,
                  semr0, semr1, semw0, semw1,
                  semi0, semi1, semi2, semi3):
    cid = lax.axis_index("c")
    sid = lax.axis_index("s")
    wid = cid * NS + sid
    idx4 = (idx0, idx1, idx2, idx3)
    rows = (rows_v0, rows_v1)
    semr = (semr0, semr1)
    semw = (semw0, semw1)
    semi = (semi0, semi1, semi2, semi3)

    def slot(c, f, p, do_w4, do_pref, do_gather):
        q, f1, f2 = 1 - p, (f + 1) % 4, (f + 2) % 4
        pltpu.make_async_copy(rows_hbm.at[idx4[f].at[0]], rows[p],
                              semr[p]).wait()
        pltpu.async_copy(rows[p], acc_sh.at[idx4[f].at[1]], semw[p],
                         add=True)
        if do_gather:
            pltpu.make_async_copy(sd_hbm.at[wid, 0], idx4[f1],
                                  semi[f1]).wait()
            if do_w4:
                pltpu.make_async_copy(rows[q], acc_sh.at[idx4[f1].at[1]],
                                      semw[q]).wait()
            pltpu.async_copy(rows_hbm.at[idx4[f1].at[0]], rows[q], semr[q])
        if do_pref:
            pltpu.async_copy(sd_hbm.at[wid, c + 2], idx4[f2], semi[f2])

    pltpu.sync_copy(sd_hbm.at[wid, 0], idx0)
    pltpu.async_copy(rows_hbm.at[idx0.at[0]], rows_v0, semr0)
    pltpu.async_copy(sd_hbm.at[wid, 1], idx1, semi1)
    pltpu.sync_copy(zeros_hbm.at[pl.ds(sid * RPS, RPS)],
                    acc_sh.at[pl.ds(sid * RPS, RPS)])
    plsc.subcore_barrier()

    slot(0, 0, 0, False, True, True)
    slot(1, 1, 1, True, True, True)
    slot(2, 2, 0, True, True, True)
    slot(3, 3, 1, True, True, True)

    def quad(g, carry):
        c0 = 4 * g
        slot(c0 + 0, 0, 0, True, True, True)
        slot(c0 + 1, 1, 1, True, True, True)
        slot(c0 + 2, 2, 0, True, True, True)
        slot(c0 + 3, 3, 1, True, True, True)
        return carry

    lax.fori_loop(1, CHW // 4 - 1, quad, 0)
    slot(CHW - 4, 0, 0, True, True, True)
    slot(CHW - 3, 1, 1, True, True, True)
    slot(CHW - 2, 2, 0, True, False, True)
    slot(CHW - 1, 3, 1, True, False, False)
    pltpu.make_async_copy(rows_v0, acc_sh.at[idx2.at[1]], semw0).wait()
    pltpu.make_async_copy(rows_v1, acc_sh.at[idx3.at[1]], semw1).wait()

    plsc.subcore_barrier()
    pltpu.sync_copy(acc_sh.at[pl.ds(sid * RPS, RPS)],
                    out_hbm.at[cid, pl.ds(sid * RPS, RPS)])


_scatter = pl.kernel(
    _scatter_body,
    out_type=jax.ShapeDtypeStruct((NC, NP, D), jnp.float32),
    mesh=_mesh,
    scratch_types=[
        pltpu.VMEM_SHARED((NP, D), jnp.float32),
        pltpu.VMEM((2, KE), jnp.int32),
        pltpu.VMEM((2, KE), jnp.int32),
        pltpu.VMEM((2, KE), jnp.int32),
        pltpu.VMEM((2, KE), jnp.int32),
        pltpu.VMEM((KE, D), jnp.float32),
        pltpu.VMEM((KE, D), jnp.float32),
        pltpu.SemaphoreType.DMA,
        pltpu.SemaphoreType.DMA,
        pltpu.SemaphoreType.DMA,
        pltpu.SemaphoreType.DMA,
        pltpu.SemaphoreType.DMA,
        pltpu.SemaphoreType.DMA,
        pltpu.SemaphoreType.DMA,
        pltpu.SemaphoreType.DMA,
    ],
)

R = 2000


def _dinv_rows(hist_ref):
    deg = hist_ref[0, :, 0] + hist_ref[1, :, 0] + 1.0
    return lax.rsqrt(deg)[:, None]


def _tc1_body(x_ref, w_ref, hist_ref, out_ref):
    h = jnp.dot(x_ref[...], w_ref[...], preferred_element_type=jnp.float32)
    out_ref[...] = h * _dinv_rows(hist_ref)


_tc1 = pl.pallas_call(
    _tc1_body,
    grid=(N // R,),
    in_specs=[
        pl.BlockSpec((R, D), lambda i: (i, 0)),
        pl.BlockSpec((D, D), lambda i: (0, 0)),
        pl.BlockSpec((NC, R, HP), lambda i: (0, i, 0)),
    ],
    out_specs=pl.BlockSpec((R, D), lambda i: (i, 0)),
    out_shape=jax.ShapeDtypeStruct((N, D), jnp.float32),
)


def _tc2_body(accp_ref, t1p_ref, hist_ref, b1_ref, gamma_ref, beta_ref,
              w2_ref, out_ref):
    dinv = _dinv_rows(hist_ref)
    acc = accp_ref[0] + accp_ref[1] + t1p_ref[...]
    conv = acc * dinv + b1_ref[...]
    s = 1.0 / jnp.sqrt(1.0 + 1e-5)
    h = jnp.maximum(conv * (gamma_ref[...] * s) + beta_ref[...], 0.0)
    out_ref[...] = jnp.dot(h, w2_ref[...],
                           preferred_element_type=jnp.float32) * dinv


_tc2 = pl.pallas_call(
    _tc2_body,
    grid=(N // R,),
    in_specs=[
        pl.BlockSpec((NC, R, D), lambda i: (0, i, 0)),
        pl.BlockSpec((R, D), lambda i: (i, 0)),
        pl.BlockSpec((NC, R, HP), lambda i: (0, i, 0)),
        pl.BlockSpec((1, D), lambda i: (0, 0)),
        pl.BlockSpec((1, D), lambda i: (0, 0)),
        pl.BlockSpec((1, D), lambda i: (0, 0)),
        pl.BlockSpec((D, D), lambda i: (0, 0)),
    ],
    out_specs=pl.BlockSpec((R, D), lambda i: (i, 0)),
    out_shape=jax.ShapeDtypeStruct((N, D), jnp.float32),
)


def _tc3_body(accp_ref, t2p_ref, hist_ref, b2_ref, x_ref, out_ref):
    dinv = _dinv_rows(hist_ref)
    acc = accp_ref[0] + accp_ref[1] + t2p_ref[...]
    out_ref[...] = acc * dinv + b2_ref[...] + x_ref[...]


_tc3 = pl.pallas_call(
    _tc3_body,
    grid=(N // R,),
    in_specs=[
        pl.BlockSpec((NC, R, D), lambda i: (0, i, 0)),
        pl.BlockSpec((R, D), lambda i: (i, 0)),
        pl.BlockSpec((NC, R, HP), lambda i: (0, i, 0)),
        pl.BlockSpec((1, D), lambda i: (0, 0)),
        pl.BlockSpec((R, D), lambda i: (i, 0)),
    ],
    out_specs=pl.BlockSpec((R, D), lambda i: (i, 0)),
    out_shape=jax.ShapeDtypeStruct((N, D), jnp.float32),
)


def kernel(x, edge_index, W1, b1, gamma, beta, W2, b2):
    src2 = edge_index[0].reshape(NW, EPW)
    dst2 = edge_index[1].reshape(NW, EPW)
    pad_s = jnp.broadcast_to(jnp.arange(PADW, dtype=jnp.int32)[None],
                             (NW, PADW))
    pad_d = jnp.broadcast_to((N + jnp.arange(PADW, dtype=jnp.int32))[None],
                             (NW, PADW))
    srcp = jnp.concatenate([src2, pad_s], axis=1).reshape(NW, CHW, KE)
    dstp = jnp.concatenate([dst2, pad_d], axis=1).reshape(NW, CHW, KE)
    sd = jnp.stack([srcp, dstp], axis=2)

    zeros_nd = jnp.zeros((NP, D), jnp.float32)
    zeros_nh = jnp.zeros((NP, HP), jnp.float32)
    ones_kh = jnp.ones((KE, HP), jnp.float32)

    hist = _hist(dstp, zeros_nh, ones_kh)
    t1p = _tc1(x, W1, hist)
    acc1 = _scatter(t1p, sd, zeros_nd)
    t2p = _tc2(acc1, t1p, hist, b1.reshape(1, D), gamma.reshape(1, D),
               beta.reshape(1, D), W2)
    acc2 = _scatter(t2p, sd, zeros_nd)
    out = _tc3(acc2, t2p, hist, b2.reshape(1, D), x)
    return out
